# baseline (speedup 1.0000x reference)
import jax
import jax.numpy as jnp
from jax import lax
from jax.experimental import pallas as pl
from jax.experimental.pallas import tpu as pltpu
import jax.experimental.pallas.tpu_sc as plsc
from functools import partial

_CH = 512
_WAVE = 16
_TC = 512

_C_T = 0
_C_U = 42
_C_V = 46
_C_AF = 50
_C_SP = 106
_C_SM = 120
_TABC = 144


def _mlp_body(sh_ref, pr_ref, we_ref, be_ref, w1h_ref, w1p_ref, b1_ref,
              w2_ref, b2_ref, wh_ref, bh_ref, axok_ref, out_ref):
    f32 = jnp.float32
    dot = partial(lax.dot_general, preferred_element_type=f32)
    mm = lambda a, b: dot(a, b, (((1,), (0,)), ((), ())))
    sh = sh_ref[...]
    pr = pr_ref[...]
    h = mm(we_ref[...], sh) + be_ref[...]
    z = mm(w1h_ref[...], h) + mm(w1p_ref[...], pr) + b1_ref[...]
    z = z * (1.0 / (1.0 + jnp.exp(-z)))
    z = mm(w2_ref[...], z) + b2_ref[...]
    z = z * (1.0 / (1.0 + jnp.exp(-z)))
    heads = jnp.tanh(mm(wh_ref[...], z) + bh_ref[...])
    iota20 = lax.broadcasted_iota(jnp.int32, pr.shape, 0)
    mx = jnp.max(pr, axis=0, keepdims=True)
    idxf = jnp.min(jnp.where(pr == mx, iota20, 127), axis=0, keepdims=True)
    onehot = (iota20 == idxf).astype(f32)
    axok = mm(axok_ref[...], onehot)
    ang = heads[0:4] * 0.5 * axok
    sf = 1.0 + 0.1 * heads[4:18]
    blk = jnp.concatenate(
        [ang, idxf.astype(f32), jnp.zeros((3, ang.shape[1]), f32), sf,
         jnp.zeros((2, ang.shape[1]), f32)], axis=0)
    out_ref[0] = blk


def _rsqrt(x):
    xi = lax.bitcast_convert_type(x, jnp.int32)
    y = lax.bitcast_convert_type(
        jnp.int32(0x5F3759DF) - (xi >> 1), jnp.float32)
    for _ in range(3):
        y = y * (1.5 - 0.5 * x * y * y)
    return y


def _sincos(x):
    x2 = x * x
    s = x * (1.0 + x2 * (-1.0 / 6.0 + x2 * (1.0 / 120.0 - x2 / 5040.0)))
    c = 1.0 + x2 * (-0.5 + x2 * (1.0 / 24.0 + x2 * (-1.0 / 720.0
                                                    + x2 / 40320.0)))
    return s, c


def _fk_body(hd_hbm, rt_hbm, tab_hbm, out_hbm, hd_v, rt_v, x_v, tab_v):
    i32 = jnp.int32
    wid = lax.axis_index("s") * 2 + lax.axis_index("c")
    pltpu.sync_copy(tab_hbm, tab_v)
    lanes = lax.broadcasted_iota(i32, (_WAVE,), 0)

    def chunk_body(j, carry):
        ch = wid * 4 + j
        pltpu.sync_copy(hd_hbm.at[ch], hd_v)
        pltpu.sync_copy(rt_hbm.at[ch], rt_v)

        def wave(w, carry2):
            base = w * _WAVE
            ds = pl.ds(base, _WAVE)
            colv = base + lanes
            topi = hd_v[4, ds].astype(i32)

            def gat(col):
                cv = jnp.full((_WAVE,), col, i32) if isinstance(col, int) \
                    else jnp.broadcast_to(col, (_WAVE,)).astype(i32)
                return plsc.load_gather(tab_v, [topi, cv])

            for c in range(3):
                for k in range(14):
                    x_v[c * 16 + k, ds] = gat(_C_T + c * 14 + k)
                x_v[c * 16 + 14, ds] = jnp.zeros((_WAVE,), jnp.float32)
                x_v[c * 16 + 15, ds] = jnp.zeros((_WAVE,), jnp.float32)

            for gi in range(4):
                ug = gat(_C_U + gi).astype(i32)
                vg = gat(_C_V + gi).astype(i32)
                pu = [plsc.load_gather(x_v, [c * 16 + ug, colv])
                      for c in range(3)]
                pv = [plsc.load_gather(x_v, [c * 16 + vg, colv])
                      for c in range(3)]
                ax = [pv[c] - pu[c] for c in range(3)]
                n2 = ax[0] * ax[0] + ax[1] * ax[1] + ax[2] * ax[2]
                nrm = n2 * _rsqrt(jnp.maximum(n2, 1e-30))
                inv = 1.0 / jnp.maximum(nrm, 1e-8)
                a = [ax[c] * inv for c in range(3)]
                ang = hd_v[gi, ds]
                sa, ca = _sincos(ang)
                omc = 1.0 - ca
                for k in range(14):
                    xk = [x_v[c * 16 + k, ds] for c in range(3)]
                    V = [xk[c] - pu[c] for c in range(3)]
                    cr = [a[1] * V[2] - a[2] * V[1],
                          a[2] * V[0] - a[0] * V[2],
                          a[0] * V[1] - a[1] * V[0]]
                    dt = a[0] * V[0] + a[1] * V[1] + a[2] * V[2]
                    t1 = dt * omc
                    m = gat(_C_AF + gi * 14 + k) > 0.5
                    for c in range(3):
                        x_v[c * 16 + k, ds] = jnp.where(
                            m, V[c] * ca + cr[c] * sa + a[c] * t1 + pu[c],
                            xk[c])

            for k in range(14):
                pidx = gat(_C_SP + k).astype(i32)
                pp = [plsc.load_gather(x_v, [c * 16 + pidx, colv])
                      for c in range(3)]
                m = gat(_C_SM + k) > 0.5
                sfk = hd_v[8 + k, ds]
                for c in range(3):
                    xk = x_v[c * 16 + k, ds]
                    x_v[c * 16 + k, ds] = jnp.where(
                        m, pp[c] + (xk - pp[c]) * sfk, xk)

            r = [rt_v[q, ds] for q in range(12)]
            for k in range(14):
                xk = [x_v[c * 16 + k, ds] for c in range(3)]
                for i in range(3):
                    x_v[i * 16 + k, ds] = (r[3 * i] * xk[0]
                                           + r[3 * i + 1] * xk[1]
                                           + r[3 * i + 2] * xk[2] + r[9 + i])
            return carry2

        lax.fori_loop(0, _CH // _WAVE, wave, 0)
        pltpu.sync_copy(x_v, out_hbm.at[ch])
        return carry

    lax.fori_loop(0, 4, chunk_body, 0)


def kernel(SH, aatype_probs, Rmats, tpos, W_e, b_e, W1, b1, W2, b2, W_t, b_t,
           W_s, b_s, template_local, template_exists, tors_axis, aff_idx,
           aff_mask, G_counts, scale_mask, scale_parent):
    f32 = jnp.float32
    B, N, DSH = SH.shape
    AAT = aatype_probs.shape[-1]
    H = W_e.shape[1]
    A14 = template_local.shape[1]
    GMAX = tors_axis.shape[1]
    BN = B * N
    NCH = BN // _CH

    shT = SH.reshape(BN, DSH).T
    prT = aatype_probs.reshape(BN, AAT).T
    rtT = jnp.concatenate([Rmats.reshape(BN, 9), tpos.reshape(BN, 3)],
                          axis=1).T
    rt3 = rtT.reshape(12, NCH, _CH).transpose(1, 0, 2)

    WeT = W_e.T
    W1hT = W1[:H].T
    W1pT = W1[H:].T
    W2T = W2.T
    WhT = jnp.concatenate([W_t, W_s], axis=1).T
    bh = jnp.concatenate([b_t, b_s])[:, None]
    beC = b_e[:, None]
    b1C = b1[:, None]
    b2C = b2[:, None]

    u = tors_axis[..., 0]
    v = tors_axis[..., 1]
    gmask = (jnp.arange(GMAX)[None, :] < G_counts[:, None])
    axokT = ((u >= 0) & (v >= 0) & gmask).astype(f32).T

    karr = jnp.arange(A14)
    ex = template_exists
    tl = template_local.transpose(0, 2, 1).reshape(AAT, 42)
    uc = jnp.clip(u, 0, None).astype(f32)
    vc = jnp.clip(v, 0, None).astype(f32)
    aic = jnp.clip(aff_idx, 0, None)
    hit = aff_mask[..., None] & (aic[..., None] == karr)
    exg = jnp.take_along_axis(ex, aic.reshape(AAT, -1), axis=1)
    exg = exg.reshape(aic.shape)
    affm = (jnp.any(hit & exg[..., None], axis=2)
            & ex[:, None, :]).astype(f32).reshape(AAT, 56)
    spc = jnp.clip(scale_parent, 0, None)
    exp_par = jnp.take_along_axis(ex, spc, axis=1)
    smf = (scale_mask & ex & exp_par).astype(f32)
    tab = jnp.concatenate(
        [tl, uc, vc, affm, spc.astype(f32), smf,
         jnp.zeros((AAT, _TABC - 134), f32)], axis=1)

    hd3 = pl.pallas_call(
        _mlp_body,
        grid=(BN // _TC,),
        in_specs=[
            pl.BlockSpec((DSH, _TC), lambda i: (0, i)),
            pl.BlockSpec((AAT, _TC), lambda i: (0, i)),
            pl.BlockSpec((H, DSH), lambda i: (0, 0)),
            pl.BlockSpec((H, 1), lambda i: (0, 0)),
            pl.BlockSpec((H, H), lambda i: (0, 0)),
            pl.BlockSpec((H, AAT), lambda i: (0, 0)),
            pl.BlockSpec((H, 1), lambda i: (0, 0)),
            pl.BlockSpec((H, H), lambda i: (0, 0)),
            pl.BlockSpec((H, 1), lambda i: (0, 0)),
            pl.BlockSpec((18, H), lambda i: (0, 0)),
            pl.BlockSpec((18, 1), lambda i: (0, 0)),
            pl.BlockSpec((GMAX, AAT), lambda i: (0, 0)),
        ],
        out_specs=pl.BlockSpec((1, 24, _TC), lambda i: (i, 0, 0)),
        out_shape=jax.ShapeDtypeStruct((BN // _TC, 24, _TC), f32),
    )(shT, prT, WeT, beC, W1hT, W1pT, b1C, W2T, b2C, WhT, bh, axokT)

    if _TC != _CH:
        hd3 = hd3.transpose(1, 0, 2).reshape(24, BN) \
                 .reshape(24, NCH, _CH).transpose(1, 0, 2)

    mesh = plsc.VectorSubcoreMesh(core_axis_name="c", subcore_axis_name="s",
                                  num_cores=2, num_subcores=16)
    fk = pl.kernel(
        _fk_body,
        out_type=jax.ShapeDtypeStruct((NCH, 48, _CH), f32),
        mesh=mesh,
        scratch_types=[
            pltpu.VMEM((24, _CH), f32),
            pltpu.VMEM((12, _CH), f32),
            pltpu.VMEM((48, _CH), f32),
            pltpu.VMEM((AAT, _TABC), f32),
        ],
        compiler_params=pltpu.CompilerParams(use_tc_tiling_on_sc=False, needs_layout_passes=False),
    )
    out3 = fk(hd3, rt3, tab)

    o = out3.transpose(1, 0, 2).reshape(48, BN)
    o = o.reshape(3, 16, BN)[:, :A14, :]
    return jnp.transpose(o, (2, 1, 0)).reshape(B, N, A14, 3)
